# baseline (speedup 1.0000x reference)
import numpy as np
import jax
import jax.numpy as jnp
from jax.experimental import pallas as pl

SLOWFAST_ALPHA = 4
HB = 4


def kernel(frames):
    C, T, H, W = frames.shape
    n = T // SLOWFAST_ALPHA
    idx = [int(v) for v in np.trunc(np.linspace(0.0, T - 1, n)).astype(np.int64)]
    HS = H // HB

    def body(x_ref, slow_ref, fast_ref):
        fast_ref[...] = x_ref[...]
        for j, t in enumerate(idx):
            slow_ref[0, j] = x_ref[0, t]

    return pl.pallas_call(
        body,
        grid=(C, HB),
        in_specs=[pl.BlockSpec((1, T, HS, W), lambda c, h: (c, 0, h, 0))],
        out_specs=[
            pl.BlockSpec((1, n, HS, W), lambda c, h: (c, 0, h, 0)),
            pl.BlockSpec((1, T, HS, W), lambda c, h: (c, 0, h, 0)),
        ],
        out_shape=[
            jax.ShapeDtypeStruct((C, n, H, W), frames.dtype),
            jax.ShapeDtypeStruct((C, T, H, W), frames.dtype),
        ],
    )(frames)


# speedup vs baseline: 1.1006x; 1.1006x over previous
import numpy as np
import jax
import jax.numpy as jnp
from jax.experimental import pallas as pl

SLOWFAST_ALPHA = 4
HB = 2


def kernel(frames):
    C, T, H, W = frames.shape
    n = T // SLOWFAST_ALPHA
    idx = [int(v) for v in np.trunc(np.linspace(0.0, T - 1, n)).astype(np.int64)]
    HS = H // HB

    def body(x_ref, slow_ref, fast_ref):
        fast_ref[...] = x_ref[...]
        for j, t in enumerate(idx):
            slow_ref[0, j] = x_ref[0, t]

    return pl.pallas_call(
        body,
        grid=(C, HB),
        in_specs=[pl.BlockSpec((1, T, HS, W), lambda c, h: (c, 0, h, 0))],
        out_specs=[
            pl.BlockSpec((1, n, HS, W), lambda c, h: (c, 0, h, 0)),
            pl.BlockSpec((1, T, HS, W), lambda c, h: (c, 0, h, 0)),
        ],
        out_shape=[
            jax.ShapeDtypeStruct((C, n, H, W), frames.dtype),
            jax.ShapeDtypeStruct((C, T, H, W), frames.dtype),
        ],
    )(frames)


# speedup vs baseline: 1.2797x; 1.1627x over previous
import numpy as np
import jax
import jax.numpy as jnp
from jax.experimental import pallas as pl

SLOWFAST_ALPHA = 4
HB = 1


def kernel(frames):
    C, T, H, W = frames.shape
    n = T // SLOWFAST_ALPHA
    idx = [int(v) for v in np.trunc(np.linspace(0.0, T - 1, n)).astype(np.int64)]
    HS = H // HB

    def body(x_ref, slow_ref, fast_ref):
        fast_ref[...] = x_ref[...]
        for j, t in enumerate(idx):
            slow_ref[0, j] = x_ref[0, t]

    return pl.pallas_call(
        body,
        grid=(C, HB),
        in_specs=[pl.BlockSpec((1, T, HS, W), lambda c, h: (c, 0, h, 0))],
        out_specs=[
            pl.BlockSpec((1, n, HS, W), lambda c, h: (c, 0, h, 0)),
            pl.BlockSpec((1, T, HS, W), lambda c, h: (c, 0, h, 0)),
        ],
        out_shape=[
            jax.ShapeDtypeStruct((C, n, H, W), frames.dtype),
            jax.ShapeDtypeStruct((C, T, H, W), frames.dtype),
        ],
    )(frames)
